# baseline (speedup 1.0000x reference)
import functools
import math

import jax
import jax.numpy as jnp
from jax import lax
from jax.experimental import pallas as pl
from jax.experimental.pallas import tpu as pltpu
from jax.experimental.pallas import tpu_sc as plsc

_NU, _NP, _NR = 50000, 10000, 100000
_H = 128
_E = 100000
_EPAD = 100096
_ET = _EPAD // 16
_B = 64
_MAXRC = 6912
_BM = 1024


def _dst_chunks(n):
    nc = max(2, 2 * math.ceil(n / (2 * _MAXRC)))
    base = math.ceil(n / nc / 16) * 16
    sizes = [base] * (nc - 1) + [n - base * (nc - 1)]
    assert all(0 < s <= _MAXRC and s % 16 == 0 for s in sizes)
    out, b = [], 0
    for s in sizes:
        out.append((b, s))
        b += s
    return out


def _node_stage(n, mode, xin, srcs, dsts):
    n_in, n_s, n_d = len(xin), len(srcs), len(dsts)
    grid = (pl.cdiv(n, _BM),)

    def body(*refs):
        i = pl.program_id(0)
        ir = refs[:n_in]
        wr = refs[n_in:n_in + 2 * (n_s + n_d)]
        orf = refs[n_in + 2 * (n_s + n_d):]
        if mode == 'lin':
            x = jnp.dot(ir[0][...], ir[1][...], preferred_element_type=jnp.float32) + ir[2][...]
        elif mode == 'relu1':
            x = jnp.maximum(ir[0][...], 0.0)
        else:
            x = jnp.maximum(ir[0][...] + ir[1][...], 0.0)
        rows = jax.lax.broadcasted_iota(jnp.int32, (_BM,), 0) + i * _BM
        valid = rows < n

        def upd_max(mref, a):
            m = jnp.max(jnp.where(valid, a, -jnp.inf))
            prev = jnp.where(i == 0, jnp.full((1, _H), -jnp.inf, jnp.float32), mref[...])
            mref[...] = jnp.maximum(prev, m)

        for j in range(n_s):
            hs = jnp.dot(x, wr[2 * j][...], preferred_element_type=jnp.float32)
            orf[3 * j][...] = hs
            a = jnp.sum(hs * wr[2 * j + 1][...], axis=1)
            orf[3 * j + 1][...] = a
            upd_max(orf[3 * j + 2], a)
        for k in range(n_d):
            wd = jnp.dot(wr[2 * n_s + 2 * k + 1][...], wr[2 * n_s + 2 * k][...],
                         preferred_element_type=jnp.float32)
            a = jnp.sum(x * wd, axis=1)
            orf[3 * n_s + 2 * k][...] = a
            upd_max(orf[3 * n_s + 2 * k + 1], a)

    full2 = pl.BlockSpec((_H, _H), lambda i: (0, 0))
    row1 = pl.BlockSpec((1, _H), lambda i: (0, 0))
    xblk = pl.BlockSpec((_BM, _H), lambda i: (i, 0))
    vblk = pl.BlockSpec((_BM,), lambda i: (i,))

    in_specs = []
    if mode == 'lin':
        in_specs += [xblk, full2, row1]
    else:
        in_specs += [xblk] * n_in
    in_specs += [full2, row1] * n_s + [full2, row1] * n_d

    out_shape, out_specs = [], []
    for _ in range(n_s):
        out_shape += [jax.ShapeDtypeStruct((n, _H), jnp.float32),
                      jax.ShapeDtypeStruct((n,), jnp.float32),
                      jax.ShapeDtypeStruct((1, _H), jnp.float32)]
        out_specs += [xblk, vblk, row1]
    for _ in range(n_d):
        out_shape += [jax.ShapeDtypeStruct((n,), jnp.float32),
                      jax.ShapeDtypeStruct((1, _H), jnp.float32)]
        out_specs += [vblk, row1]

    f = pl.pallas_call(body, grid=grid, in_specs=in_specs,
                       out_specs=out_specs, out_shape=out_shape)
    args = list(xin)
    for W, av in srcs:
        args += [W, av]
    for WT, av in dsts:
        args += [WT, av]
    return f(*args)


def _add2(a, b):
    n = a.shape[0]
    blk = pl.BlockSpec((_BM, _H), lambda i: (i, 0))

    def body(ar, br, orf):
        orf[...] = ar[...] + br[...]

    return pl.pallas_call(body, grid=(pl.cdiv(n, _BM),),
                          in_specs=[blk, blk], out_specs=blk,
                          out_shape=jax.ShapeDtypeStruct((n, _H), jnp.float32))(a, b)


_RCPAD = ((_MAXRC + _B - 1) // _B) * _B


@functools.cache
def _make_agg(n_src, n_dst):
    chunks = _dst_chunks(n_dst)
    mesh = plsc.VectorSubcoreMesh(core_axis_name="c", subcore_axis_name="s")

    @functools.partial(
        pl.kernel, mesh=mesh,
        compiler_params=pltpu.CompilerParams(needs_layout_passes=False),
        out_type=jax.ShapeDtypeStruct((n_dst, _H), jnp.float32),
        scratch_types=[
            pltpu.VMEM_SHARED((_RCPAD, _H), jnp.float32),
            pltpu.VMEM_SHARED((16 * _RCPAD,), jnp.float32),
            pltpu.VMEM_SHARED((_RCPAD,), jnp.float32),
            pltpu.VMEM((_ET + 16,), jnp.int32),
            pltpu.VMEM((_ET + 16,), jnp.int32),
            pltpu.VMEM((_ET + 16,), jnp.float32),
            pltpu.VMEM((_ET + 144,), jnp.int32),
            pltpu.VMEM((_ET + 144,), jnp.int32),
            pltpu.VMEM((_ET + 144,), jnp.float32),
            pltpu.VMEM((_B,), jnp.int32),
            pltpu.VMEM((_B,), jnp.int32),
            pltpu.VMEM((_RCPAD,), jnp.float32),
            pltpu.VMEM((_B, _H), jnp.float32),
            pltpu.VMEM((_B, _H), jnp.float32),
            pltpu.VMEM((_B,), jnp.float32),
            pltpu.VMEM((_B,), jnp.float32),
            pltpu.VMEM((_RCPAD // 16,), jnp.float32),
            pltpu.VMEM((_H,), jnp.float32),
            pltpu.VMEM((_H,), jnp.float32),
            pltpu.VMEM((_H,), jnp.float32),
            pltpu.SemaphoreType.DMA,
            pltpu.SemaphoreType.DMA,
        ],
    )
    def agg(hs, asv, adv, mxs, mxd, bias, srcp, dstp, out,
            spacc, spdenp, spdenF, src_b, dst_b, ex_b,
            srcm_b, dstl_b, exm_b, six, dix, denp, hsb, msgb,
            den_s, den_b, den_r, bias_b, ms_b, md_b, sem, sem2):
        c = lax.axis_index("c")
        s = lax.axis_index("s")
        lanes = lax.broadcasted_iota(jnp.int32, (16,), 0)
        zf = jnp.zeros((16,), jnp.float32)
        zi = jnp.zeros((16,), jnp.int32)

        e0 = s * _ET
        pltpu.sync_copy(srcp.at[pl.ds(e0, _ET)], src_b.at[pl.ds(0, _ET)])
        pltpu.sync_copy(dstp.at[pl.ds(e0, _ET)], dst_b.at[pl.ds(0, _ET)])
        pltpu.sync_copy(mxs.at[0], ms_b)
        pltpu.sync_copy(mxd.at[0], md_b)
        pltpu.sync_copy(bias, bias_b)
        pltpu.async_copy(asv.at[src_b.at[pl.ds(0, _ET)]], ex_b.at[pl.ds(0, _ET)], sem)
        pltpu.async_copy(adv.at[dst_b.at[pl.ds(0, _ET)]], exm_b.at[pl.ds(0, _ET)], sem2)
        pltpu.make_async_copy(asv.at[src_b.at[pl.ds(0, _ET)]], ex_b.at[pl.ds(0, _ET)], sem).wait()
        pltpu.make_async_copy(adv.at[dst_b.at[pl.ds(0, _ET)]], exm_b.at[pl.ds(0, _ET)], sem2).wait()

        msum = ms_b[pl.ds(0, 16)] + md_b[pl.ds(0, 16)]
        cshift = jnp.where(msum > 0, msum, 0.2 * msum)
        bvecs = [bias_b[pl.ds(16 * k, 16)] for k in range(8)]

        def ex_body(i, _):
            a = ex_b[pl.ds(i * 16, 16)] + exm_b[pl.ds(i * 16, 16)]
            al = jnp.where(a > 0, a, 0.2 * a)
            gi = e0 + i * 16 + lanes
            ex_b[pl.ds(i * 16, 16)] = jnp.where(gi < _E, jnp.exp(al - cshift), 0.0)
            return 0

        lax.fori_loop(0, _ET // 16, ex_body, 0)

        ubase = chunks[0][1]

        def filt_body(i, fcnt):
            d = dst_b[pl.ds(i * 16, 16)]
            m = ((d // ubase) % 2) == c
            plsc.store_compressed(src_b.at[pl.ds(fcnt, 16)],
                                  src_b[pl.ds(i * 16, 16)], mask=m)
            plsc.store_compressed(dst_b.at[pl.ds(fcnt, 16)], d, mask=m)
            plsc.store_compressed(ex_b.at[pl.ds(fcnt, 16)],
                                  ex_b[pl.ds(i * 16, 16)], mask=m)
            return fcnt + jnp.sum(m.astype(jnp.int32))

        fcnt = lax.fori_loop(0, _ET // 16, filt_body, jnp.int32(0))
        dst_b[pl.ds(fcnt, 16)] = jnp.full((16,), -1, jnp.int32)
        nscan = (fcnt + 15) // 16

        def chunk_pass(cb, csz):
            nfull, prem = csz // _B, csz % _B
            nround = (nfull + 15) // 16
            if True:
                def mz_body(i, _):
                    msgb[i // 8, pl.ds((i % 8) * 16, 16)] = zf
                    return 0

                lax.fori_loop(0, _B * 8, mz_body, 0)

                def zero_blk(t, _):
                    z = t * 16 + s

                    @pl.when(z < nfull)
                    def _():
                        pltpu.sync_copy(msgb, spacc.at[pl.ds(z * _B, _B)])
                    return 0

                lax.fori_loop(0, nround, zero_blk, 0)
                if prem:
                    @pl.when(s == nfull % 16)
                    def _():
                        pltpu.sync_copy(msgb.at[pl.ds(0, prem)],
                                        spacc.at[pl.ds(nfull * _B, prem)])

                def dz_body(i, _):
                    denp[pl.ds(i * 16, 16)] = zf
                    return 0

                lax.fori_loop(0, _RCPAD // 16, dz_body, 0)
                plsc.subcore_barrier()

                def scan_body(i, cnt):
                    d = dst_b[pl.ds(i * 16, 16)]
                    ex = ex_b[pl.ds(i * 16, 16)]
                    m = (d >= cb) & (d < cb + csz)
                    dl = d - cb
                    plsc.addupdate_scatter(denp, [dl], ex, mask=m)
                    plsc.store_compressed(srcm_b.at[pl.ds(cnt, 16)],
                                          src_b[pl.ds(i * 16, 16)], mask=m)
                    plsc.store_compressed(dstl_b.at[pl.ds(cnt, 16)], dl, mask=m)
                    plsc.store_compressed(exm_b.at[pl.ds(cnt, 16)], ex, mask=m)
                    return cnt + jnp.sum(m.astype(jnp.int32))

                cnt = lax.fori_loop(0, nscan, scan_body, jnp.int32(0))
                pltpu.sync_copy(denp, spdenp.at[pl.ds(s * _RCPAD, _RCPAD)])

                def pad_body(k, _):
                    srcm_b[pl.ds(cnt + k * 16, 16)] = zi
                    dstl_b[pl.ds(cnt + k * 16, 16)] = zi
                    exm_b[pl.ds(cnt + k * 16, 16)] = zf
                    return 0

                lax.fori_loop(0, _B // 16, pad_body, 0)
                nb = (cnt + _B - 1) // _B

                def batch_body(bi, _):
                    o = bi * _B
                    for k in range(_B // 16):
                        six[pl.ds(k * 16, 16)] = srcm_b[pl.ds(o + k * 16, 16)]
                        dix[pl.ds(k * 16, 16)] = dstl_b[pl.ds(o + k * 16, 16)]
                    pltpu.async_copy(hs.at[six], hsb, sem).wait()

                    def row_body(r, _):
                        exsp = plsc.load_gather(exm_b, [jnp.full((16,), o + r, jnp.int32)])
                        for k in range(8):
                            msgb[r, pl.ds(k * 16, 16)] = hsb[r, pl.ds(k * 16, 16)] * exsp
                        return 0

                    lax.fori_loop(0, _B, row_body, 0)
                    pltpu.sync_copy(msgb, spacc.at[dix], add=True)
                    return 0

                lax.fori_loop(0, nb, batch_body, 0)
                plsc.subcore_barrier()

                rs = _RCPAD // 16
                d0 = s * rs

                def dred(t, _):
                    pltpu.sync_copy(spdenp.at[pl.ds(t * _RCPAD + d0, rs)], den_r)

                    def dacc(j, _):
                        cur = den_r[pl.ds(j * 16, 16)]
                        prev = denp[pl.ds(j * 16, 16)]
                        denp[pl.ds(j * 16, 16)] = jnp.where(t == 0, cur, prev + cur)
                        return 0

                    lax.fori_loop(0, rs // 16, dacc, 0)
                    return 0

                lax.fori_loop(0, 16, dred, 0)
                pltpu.sync_copy(denp.at[pl.ds(0, rs)], spdenF.at[pl.ds(d0, rs)])
                plsc.subcore_barrier()

                def flush_blk(z, rows):
                    pltpu.sync_copy(spacc.at[pl.ds(z * _B, rows)],
                                    hsb.at[pl.ds(0, rows)])
                    pltpu.sync_copy(spdenF.at[pl.ds(z * _B, rows)],
                                    den_s.at[pl.ds(0, rows)])
                    for k in range(rows // 16):
                        den_b[pl.ds(k * 16, 16)] = (
                            1.0 / (den_s[pl.ds(k * 16, 16)] + 1e-16))

                    def f_body(r, _):
                        rc = plsc.load_gather(den_b, [jnp.full((16,), r, jnp.int32)])
                        for k in range(8):
                            msgb[r, pl.ds(k * 16, 16)] = (
                                hsb[r, pl.ds(k * 16, 16)] * rc + bvecs[k])
                        return 0

                    lax.fori_loop(0, rows, f_body, 0)
                    pltpu.sync_copy(msgb.at[pl.ds(0, rows)],
                                    out.at[pl.ds(cb + z * _B, rows)])

                def flush_round(t, _):
                    z = t * 16 + s

                    @pl.when(z < nfull)
                    def _():
                        flush_blk(z, _B)
                    return 0

                lax.fori_loop(0, nround, flush_round, 0)
                if prem:
                    @pl.when(s == nfull % 16)
                    def _():
                        flush_blk(jnp.int32(nfull), prem)
                plsc.subcore_barrier()

        nu = len(chunks) - 1
        nmy = (nu - c + 1) // 2

        def uni_body(t, _):
            cb = pl.multiple_of((2 * t + c) * ubase, 16)
            chunk_pass(cb, ubase)
            return 0

        lax.fori_loop(0, nmy, uni_body, 0)

        @pl.when(c == (nu % 2))
        def _():
            chunk_pass(chunks[-1][0], chunks[-1][1])

    return agg


def kernel(x_user, x_product, x_review, params,
           edge_index_writes, edge_index_reviews,
           edge_index_rev_by, edge_index_written_by):
    pads = _EPAD - _E

    def prep_edges(ei):
        return (jnp.pad(ei[0], (0, pads)), jnp.pad(ei[1], (0, pads)))

    e_w = prep_edges(edge_index_writes)
    e_rv = prep_edges(edge_index_reviews)
    e_rb = prep_edges(edge_index_rev_by)
    e_wb = prep_edges(edge_index_written_by)

    agg_u2r = _make_agg(_NU, _NR)
    agg_p2r = _make_agg(_NP, _NR)
    agg_r2p = _make_agg(_NR, _NP)
    agg_r2u = _make_agg(_NR, _NU)

    def srcp(cp):
        return (cp['Ws'], cp['as'][None])

    def dstp(cp):
        return (cp['Wd'].T, cp['ad'][None])

    def layer(cp, mode_u, xin_u, mode_p, xin_p, mode_r, xin_r):
        hs_w, as_w, ms_w, ad_wb, md_wb = _node_stage(
            _NU, mode_u, xin_u, [srcp(cp['writes'])], [dstp(cp['written_by'])])
        hs_rb, as_rb, ms_rb, ad_rv, md_rv = _node_stage(
            _NP, mode_p, xin_p, [srcp(cp['rev_by'])], [dstp(cp['reviews'])])
        (hs_rv, as_rv, ms_rv, hs_wb, as_wb, ms_wb,
         ad_w, md_w, ad_rb, md_rb) = _node_stage(
            _NR, mode_r, xin_r,
            [srcp(cp['reviews']), srcp(cp['written_by'])],
            [dstp(cp['writes']), dstp(cp['rev_by'])])

        o_w = agg_u2r(hs_w, as_w, ad_w, ms_w, md_w, cp['writes']['b'], *e_w)
        o_rb = agg_p2r(hs_rb, as_rb, ad_rb, ms_rb, md_rb, cp['rev_by']['b'], *e_rb)
        o_rv = agg_r2p(hs_rv, as_rv, ad_rv, ms_rv, md_rv, cp['reviews']['b'], *e_rv)
        o_wb = agg_r2u(hs_wb, as_wb, ad_wb, ms_wb, md_wb, cp['written_by']['b'], *e_wb)
        return o_w, o_rb, o_rv, o_wb

    o_w1, o_rb1, o_rv1, o_wb1 = layer(
        params['conv1'],
        'lin', (x_user, params['lin_user_W'], params['lin_user_b'][None]),
        'lin', (x_product, params['lin_product_W'], params['lin_product_b'][None]),
        'lin', (x_review, params['lin_review_W'], params['lin_review_b'][None]))

    o_w2, o_rb2, o_rv2, o_wb2 = layer(
        params['conv2'],
        'relu1', (o_wb1,), 'relu1', (o_rv1,), 'relu2', (o_w1, o_rb1))

    return o_wb2, o_rv2, _add2(o_w2, o_rb2)


# speedup vs baseline: 1.2828x; 1.2828x over previous
import functools
import math

import jax
import jax.numpy as jnp
from jax import lax
from jax.experimental import pallas as pl
from jax.experimental.pallas import tpu as pltpu
from jax.experimental.pallas import tpu_sc as plsc

_NU, _NP, _NR = 50000, 10000, 100000
_H = 128
_E = 100000
_EPAD = 100096
_ET = _EPAD // 16
_B = 64
_MAXRC = 6912
_BM = 1024


def _dst_chunks(n):
    nc = max(2, 2 * math.ceil(n / (2 * _MAXRC)))
    base = math.ceil(n / nc / 16) * 16
    sizes = [base] * (nc - 1) + [n - base * (nc - 1)]
    assert all(0 < s <= _MAXRC and s % 16 == 0 for s in sizes)
    out, b = [], 0
    for s in sizes:
        out.append((b, s))
        b += s
    return out


def _node_stage(n, mode, xin, srcs, dsts):
    n_in, n_s, n_d = len(xin), len(srcs), len(dsts)
    grid = (pl.cdiv(n, _BM),)

    def body(*refs):
        i = pl.program_id(0)
        ir = refs[:n_in]
        wr = refs[n_in:n_in + 2 * (n_s + n_d)]
        orf = refs[n_in + 2 * (n_s + n_d):]
        if mode == 'lin':
            x = jnp.dot(ir[0][...], ir[1][...], preferred_element_type=jnp.float32) + ir[2][...]
        elif mode == 'relu1':
            x = jnp.maximum(ir[0][...], 0.0)
        else:
            x = jnp.maximum(ir[0][...] + ir[1][...], 0.0)
        rows = jax.lax.broadcasted_iota(jnp.int32, (_BM,), 0) + i * _BM
        valid = rows < n

        def upd_max(mref, a):
            m = jnp.max(jnp.where(valid, a, -jnp.inf))
            prev = jnp.where(i == 0, jnp.full((1, _H), -jnp.inf, jnp.float32), mref[...])
            mref[...] = jnp.maximum(prev, m)

        for j in range(n_s):
            hs = jnp.dot(x, wr[2 * j][...], preferred_element_type=jnp.float32)
            orf[3 * j][...] = hs
            a = jnp.sum(hs * wr[2 * j + 1][...], axis=1)
            orf[3 * j + 1][...] = a
            upd_max(orf[3 * j + 2], a)
        for k in range(n_d):
            wd = jnp.dot(wr[2 * n_s + 2 * k + 1][...], wr[2 * n_s + 2 * k][...],
                         preferred_element_type=jnp.float32)
            a = jnp.sum(x * wd, axis=1)
            orf[3 * n_s + 2 * k][...] = a
            upd_max(orf[3 * n_s + 2 * k + 1], a)

    full2 = pl.BlockSpec((_H, _H), lambda i: (0, 0))
    row1 = pl.BlockSpec((1, _H), lambda i: (0, 0))
    xblk = pl.BlockSpec((_BM, _H), lambda i: (i, 0))
    vblk = pl.BlockSpec((_BM,), lambda i: (i,))

    in_specs = []
    if mode == 'lin':
        in_specs += [xblk, full2, row1]
    else:
        in_specs += [xblk] * n_in
    in_specs += [full2, row1] * n_s + [full2, row1] * n_d

    out_shape, out_specs = [], []
    for _ in range(n_s):
        out_shape += [jax.ShapeDtypeStruct((n, _H), jnp.float32),
                      jax.ShapeDtypeStruct((n,), jnp.float32),
                      jax.ShapeDtypeStruct((1, _H), jnp.float32)]
        out_specs += [xblk, vblk, row1]
    for _ in range(n_d):
        out_shape += [jax.ShapeDtypeStruct((n,), jnp.float32),
                      jax.ShapeDtypeStruct((1, _H), jnp.float32)]
        out_specs += [vblk, row1]

    f = pl.pallas_call(body, grid=grid, in_specs=in_specs,
                       out_specs=out_specs, out_shape=out_shape)
    args = list(xin)
    for W, av in srcs:
        args += [W, av]
    for WT, av in dsts:
        args += [WT, av]
    return f(*args)


def _add2(a, b):
    n = a.shape[0]
    blk = pl.BlockSpec((_BM, _H), lambda i: (i, 0))

    def body(ar, br, orf):
        orf[...] = ar[...] + br[...]

    return pl.pallas_call(body, grid=(pl.cdiv(n, _BM),),
                          in_specs=[blk, blk], out_specs=blk,
                          out_shape=jax.ShapeDtypeStruct((n, _H), jnp.float32))(a, b)


_RCPAD = ((_MAXRC + _B - 1) // _B) * _B


@functools.cache
def _make_agg(n_src, n_dst):
    chunks = _dst_chunks(n_dst)
    mesh = plsc.VectorSubcoreMesh(core_axis_name="c", subcore_axis_name="s")

    @functools.partial(
        pl.kernel, mesh=mesh,
        compiler_params=pltpu.CompilerParams(needs_layout_passes=False),
        out_type=jax.ShapeDtypeStruct((n_dst, _H), jnp.float32),
        scratch_types=[
            pltpu.VMEM_SHARED((_RCPAD, _H), jnp.float32),
            pltpu.VMEM_SHARED((16 * _RCPAD,), jnp.float32),
            pltpu.VMEM_SHARED((_RCPAD,), jnp.float32),
            pltpu.VMEM((_ET + 16,), jnp.int32),
            pltpu.VMEM((_ET + 16,), jnp.int32),
            pltpu.VMEM((_ET + 16,), jnp.float32),
            pltpu.VMEM((_ET + 144,), jnp.int32),
            pltpu.VMEM((_ET + 144,), jnp.int32),
            pltpu.VMEM((_ET + 144,), jnp.float32),
            pltpu.VMEM((_B,), jnp.int32),
            pltpu.VMEM((_B,), jnp.int32),
            pltpu.VMEM((_RCPAD,), jnp.float32),
            pltpu.VMEM((_B, _H), jnp.float32),
            pltpu.VMEM((_B, _H), jnp.float32),
            pltpu.VMEM((_B,), jnp.float32),
            pltpu.VMEM((_B,), jnp.float32),
            pltpu.VMEM((_RCPAD // 16,), jnp.float32),
            pltpu.VMEM((_H,), jnp.float32),
            pltpu.VMEM((_H,), jnp.float32),
            pltpu.VMEM((_H,), jnp.float32),
            pltpu.SemaphoreType.DMA,
            pltpu.SemaphoreType.DMA,
        ],
    )
    def agg(hs, asv, adv, mxs, mxd, bias, srcp, dstp, out,
            spacc, spdenp, spdenF, src_b, dst_b, ex_b,
            srcm_b, dstl_b, exm_b, six, dix, denp, hsb, msgb,
            den_s, den_b, den_r, bias_b, ms_b, md_b, sem, sem2):
        c = lax.axis_index("c")
        s = lax.axis_index("s")
        lanes = lax.broadcasted_iota(jnp.int32, (16,), 0)
        dnums = lax.GatherDimensionNumbers(
            offset_dims=(), collapsed_slice_dims=(0,), start_index_map=(0,))

        def lane_bcast(v, j):
            idx = jnp.full((16, 1), j, jnp.int32)
            return lax.gather(v, idx, dnums, slice_sizes=(1,),
                              mode=lax.GatherScatterMode.PROMISE_IN_BOUNDS)
        zf = jnp.zeros((16,), jnp.float32)
        zi = jnp.zeros((16,), jnp.int32)

        e0 = s * _ET
        pltpu.sync_copy(srcp.at[pl.ds(e0, _ET)], src_b.at[pl.ds(0, _ET)])
        pltpu.sync_copy(dstp.at[pl.ds(e0, _ET)], dst_b.at[pl.ds(0, _ET)])
        pltpu.sync_copy(mxs.at[0], ms_b)
        pltpu.sync_copy(mxd.at[0], md_b)
        pltpu.sync_copy(bias, bias_b)
        pltpu.async_copy(asv.at[src_b.at[pl.ds(0, _ET)]], ex_b.at[pl.ds(0, _ET)], sem)
        pltpu.async_copy(adv.at[dst_b.at[pl.ds(0, _ET)]], exm_b.at[pl.ds(0, _ET)], sem2)
        pltpu.make_async_copy(asv.at[src_b.at[pl.ds(0, _ET)]], ex_b.at[pl.ds(0, _ET)], sem).wait()
        pltpu.make_async_copy(adv.at[dst_b.at[pl.ds(0, _ET)]], exm_b.at[pl.ds(0, _ET)], sem2).wait()

        msum = ms_b[pl.ds(0, 16)] + md_b[pl.ds(0, 16)]
        cshift = jnp.where(msum > 0, msum, 0.2 * msum)
        bvecs = [bias_b[pl.ds(16 * k, 16)] for k in range(8)]

        def ex_body(i, _):
            a = ex_b[pl.ds(i * 16, 16)] + exm_b[pl.ds(i * 16, 16)]
            al = jnp.where(a > 0, a, 0.2 * a)
            gi = e0 + i * 16 + lanes
            ex_b[pl.ds(i * 16, 16)] = jnp.where(gi < _E, jnp.exp(al - cshift), 0.0)
            return 0

        lax.fori_loop(0, _ET // 16, ex_body, 0)

        def chunk_pass(cb, csz):
            nfull, prem = csz // _B, csz % _B
            nround = (nfull + 15) // 16
            if True:
                def mz_body(i, _):
                    msgb[i // 8, pl.ds((i % 8) * 16, 16)] = zf
                    return 0

                lax.fori_loop(0, _B * 8, mz_body, 0)

                def zero_blk(t, _):
                    z = t * 16 + s

                    @pl.when(z < nfull)
                    def _():
                        pltpu.sync_copy(msgb, spacc.at[pl.ds(z * _B, _B)])
                    return 0

                lax.fori_loop(0, nround, zero_blk, 0)
                if prem:
                    @pl.when(s == nfull % 16)
                    def _():
                        pltpu.sync_copy(msgb.at[pl.ds(0, prem)],
                                        spacc.at[pl.ds(nfull * _B, prem)])

                def dz_body(i, _):
                    denp[pl.ds(i * 16, 16)] = zf
                    return 0

                lax.fori_loop(0, _RCPAD // 16, dz_body, 0)
                plsc.subcore_barrier()

                def scan_body(i, cnt):
                    d = dst_b[pl.ds(i * 16, 16)]
                    ex = ex_b[pl.ds(i * 16, 16)]
                    m = (d >= cb) & (d < cb + csz)
                    dl = d - cb
                    plsc.addupdate_scatter(denp, [dl], ex, mask=m)
                    plsc.store_compressed(srcm_b.at[pl.ds(cnt, 16)],
                                          src_b[pl.ds(i * 16, 16)], mask=m)
                    plsc.store_compressed(dstl_b.at[pl.ds(cnt, 16)], dl, mask=m)
                    plsc.store_compressed(exm_b.at[pl.ds(cnt, 16)], ex, mask=m)
                    return cnt + jnp.sum(m.astype(jnp.int32))

                cnt = lax.fori_loop(0, _ET // 16, scan_body, jnp.int32(0))
                pltpu.sync_copy(denp, spdenp.at[pl.ds(s * _RCPAD, _RCPAD)])

                def pad_body(k, _):
                    srcm_b[pl.ds(cnt + k * 16, 16)] = zi
                    dstl_b[pl.ds(cnt + k * 16, 16)] = zi
                    exm_b[pl.ds(cnt + k * 16, 16)] = zf
                    return 0

                lax.fori_loop(0, _B // 16, pad_body, 0)
                nb = (cnt + _B - 1) // _B

                def batch_body(bi, _):
                    o = bi * _B
                    for k in range(_B // 16):
                        six[pl.ds(k * 16, 16)] = srcm_b[pl.ds(o + k * 16, 16)]
                        dix[pl.ds(k * 16, 16)] = dstl_b[pl.ds(o + k * 16, 16)]
                    pltpu.async_copy(hs.at[six], hsb, sem).wait()

                    for g in range(_B // 16):
                        exv = exm_b[pl.ds(o + g * 16, 16)]
                        for j in range(16):
                            exsp = lane_bcast(exv, j)
                            r = g * 16 + j
                            for k in range(8):
                                msgb[r, pl.ds(k * 16, 16)] = (
                                    hsb[r, pl.ds(k * 16, 16)] * exsp)
                    pltpu.sync_copy(msgb, spacc.at[dix], add=True)
                    return 0

                lax.fori_loop(0, nb, batch_body, 0)
                plsc.subcore_barrier()

                rs = _RCPAD // 16
                d0 = s * rs

                def dred(t, _):
                    pltpu.sync_copy(spdenp.at[pl.ds(t * _RCPAD + d0, rs)], den_r)

                    def dacc(j, _):
                        cur = den_r[pl.ds(j * 16, 16)]
                        prev = denp[pl.ds(j * 16, 16)]
                        denp[pl.ds(j * 16, 16)] = jnp.where(t == 0, cur, prev + cur)
                        return 0

                    lax.fori_loop(0, rs // 16, dacc, 0)
                    return 0

                lax.fori_loop(0, 16, dred, 0)
                pltpu.sync_copy(denp.at[pl.ds(0, rs)], spdenF.at[pl.ds(d0, rs)])
                plsc.subcore_barrier()

                def flush_blk(z, rows):
                    pltpu.sync_copy(spacc.at[pl.ds(z * _B, rows)],
                                    hsb.at[pl.ds(0, rows)])
                    pltpu.sync_copy(spdenF.at[pl.ds(z * _B, rows)],
                                    den_s.at[pl.ds(0, rows)])
                    for k in range(rows // 16):
                        den_b[pl.ds(k * 16, 16)] = (
                            1.0 / (den_s[pl.ds(k * 16, 16)] + 1e-16))

                    for g in range(rows // 16):
                        rcv = den_b[pl.ds(g * 16, 16)]
                        for j in range(16):
                            rc = lane_bcast(rcv, j)
                            r = g * 16 + j
                            for k in range(8):
                                msgb[r, pl.ds(k * 16, 16)] = (
                                    hsb[r, pl.ds(k * 16, 16)] * rc + bvecs[k])
                    pltpu.sync_copy(msgb.at[pl.ds(0, rows)],
                                    out.at[pl.ds(cb + z * _B, rows)])

                def flush_round(t, _):
                    z = t * 16 + s

                    @pl.when(z < nfull)
                    def _():
                        flush_blk(z, _B)
                    return 0

                lax.fori_loop(0, nround, flush_round, 0)
                if prem:
                    @pl.when(s == nfull % 16)
                    def _():
                        flush_blk(jnp.int32(nfull), prem)
                plsc.subcore_barrier()

        nu = len(chunks) - 1
        ubase = chunks[0][1]
        nmy = (nu - c + 1) // 2

        def uni_body(t, _):
            cb = pl.multiple_of((2 * t + c) * ubase, 16)
            chunk_pass(cb, ubase)
            return 0

        lax.fori_loop(0, nmy, uni_body, 0)

        @pl.when(c == (nu % 2))
        def _():
            chunk_pass(chunks[-1][0], chunks[-1][1])

    return agg


def kernel(x_user, x_product, x_review, params,
           edge_index_writes, edge_index_reviews,
           edge_index_rev_by, edge_index_written_by):
    pads = _EPAD - _E

    def prep_edges(ei):
        return (jnp.pad(ei[0], (0, pads)), jnp.pad(ei[1], (0, pads)))

    e_w = prep_edges(edge_index_writes)
    e_rv = prep_edges(edge_index_reviews)
    e_rb = prep_edges(edge_index_rev_by)
    e_wb = prep_edges(edge_index_written_by)

    agg_u2r = _make_agg(_NU, _NR)
    agg_p2r = _make_agg(_NP, _NR)
    agg_r2p = _make_agg(_NR, _NP)
    agg_r2u = _make_agg(_NR, _NU)

    def srcp(cp):
        return (cp['Ws'], cp['as'][None])

    def dstp(cp):
        return (cp['Wd'].T, cp['ad'][None])

    def layer(cp, mode_u, xin_u, mode_p, xin_p, mode_r, xin_r):
        hs_w, as_w, ms_w, ad_wb, md_wb = _node_stage(
            _NU, mode_u, xin_u, [srcp(cp['writes'])], [dstp(cp['written_by'])])
        hs_rb, as_rb, ms_rb, ad_rv, md_rv = _node_stage(
            _NP, mode_p, xin_p, [srcp(cp['rev_by'])], [dstp(cp['reviews'])])
        (hs_rv, as_rv, ms_rv, hs_wb, as_wb, ms_wb,
         ad_w, md_w, ad_rb, md_rb) = _node_stage(
            _NR, mode_r, xin_r,
            [srcp(cp['reviews']), srcp(cp['written_by'])],
            [dstp(cp['writes']), dstp(cp['rev_by'])])

        o_w = agg_u2r(hs_w, as_w, ad_w, ms_w, md_w, cp['writes']['b'], *e_w)
        o_rb = agg_p2r(hs_rb, as_rb, ad_rb, ms_rb, md_rb, cp['rev_by']['b'], *e_rb)
        o_rv = agg_r2p(hs_rv, as_rv, ad_rv, ms_rv, md_rv, cp['reviews']['b'], *e_rv)
        o_wb = agg_r2u(hs_wb, as_wb, ad_wb, ms_wb, md_wb, cp['written_by']['b'], *e_wb)
        return o_w, o_rb, o_rv, o_wb

    o_w1, o_rb1, o_rv1, o_wb1 = layer(
        params['conv1'],
        'lin', (x_user, params['lin_user_W'], params['lin_user_b'][None]),
        'lin', (x_product, params['lin_product_W'], params['lin_product_b'][None]),
        'lin', (x_review, params['lin_review_W'], params['lin_review_b'][None]))

    o_w2, o_rb2, o_rv2, o_wb2 = layer(
        params['conv2'],
        'relu1', (o_wb1,), 'relu1', (o_rv1,), 'relu2', (o_w1, o_rb1))

    return o_wb2, o_rv2, _add2(o_w2, o_rb2)


# speedup vs baseline: 1.3081x; 1.0198x over previous
import functools
import math

import jax
import jax.numpy as jnp
from jax import lax
from jax.experimental import pallas as pl
from jax.experimental.pallas import tpu as pltpu
from jax.experimental.pallas import tpu_sc as plsc

_NU, _NP, _NR = 50000, 10000, 100000
_H = 128
_E = 100000
_EPAD = 100096
_ET = _EPAD // 16
_B = 64
_MAXRC = 6912
_BM = 1024


def _dst_chunks(n):
    nc = max(2, 2 * math.ceil(n / (2 * _MAXRC)))
    base = math.ceil(n / nc / 16) * 16
    sizes = [base] * (nc - 1) + [n - base * (nc - 1)]
    assert all(0 < s <= _MAXRC and s % 16 == 0 for s in sizes)
    out, b = [], 0
    for s in sizes:
        out.append((b, s))
        b += s
    return out


def _node_stage(n, mode, xin, srcs, dsts):
    n_in, n_s, n_d = len(xin), len(srcs), len(dsts)
    grid = (pl.cdiv(n, _BM),)

    def body(*refs):
        i = pl.program_id(0)
        ir = refs[:n_in]
        wr = refs[n_in:n_in + 2 * (n_s + n_d)]
        orf = refs[n_in + 2 * (n_s + n_d):]
        if mode == 'lin':
            x = jnp.dot(ir[0][...], ir[1][...], preferred_element_type=jnp.float32) + ir[2][...]
        elif mode == 'relu1':
            x = jnp.maximum(ir[0][...], 0.0)
        else:
            x = jnp.maximum(ir[0][...] + ir[1][...], 0.0)
        rows = jax.lax.broadcasted_iota(jnp.int32, (_BM,), 0) + i * _BM
        valid = rows < n

        def upd_max(mref, a):
            m = jnp.max(jnp.where(valid, a, -jnp.inf))
            prev = jnp.where(i == 0, jnp.full((1, _H), -jnp.inf, jnp.float32), mref[...])
            mref[...] = jnp.maximum(prev, m)

        for j in range(n_s):
            hs = jnp.dot(x, wr[2 * j][...], preferred_element_type=jnp.float32)
            orf[3 * j][...] = hs
            a = jnp.sum(hs * wr[2 * j + 1][...], axis=1)
            orf[3 * j + 1][...] = a
            upd_max(orf[3 * j + 2], a)
        for k in range(n_d):
            wd = jnp.dot(wr[2 * n_s + 2 * k + 1][...], wr[2 * n_s + 2 * k][...],
                         preferred_element_type=jnp.float32)
            a = jnp.sum(x * wd, axis=1)
            orf[3 * n_s + 2 * k][...] = a
            upd_max(orf[3 * n_s + 2 * k + 1], a)

    full2 = pl.BlockSpec((_H, _H), lambda i: (0, 0))
    row1 = pl.BlockSpec((1, _H), lambda i: (0, 0))
    xblk = pl.BlockSpec((_BM, _H), lambda i: (i, 0))
    vblk = pl.BlockSpec((_BM,), lambda i: (i,))

    in_specs = []
    if mode == 'lin':
        in_specs += [xblk, full2, row1]
    else:
        in_specs += [xblk] * n_in
    in_specs += [full2, row1] * n_s + [full2, row1] * n_d

    out_shape, out_specs = [], []
    for _ in range(n_s):
        out_shape += [jax.ShapeDtypeStruct((n, _H), jnp.float32),
                      jax.ShapeDtypeStruct((n,), jnp.float32),
                      jax.ShapeDtypeStruct((1, _H), jnp.float32)]
        out_specs += [xblk, vblk, row1]
    for _ in range(n_d):
        out_shape += [jax.ShapeDtypeStruct((n,), jnp.float32),
                      jax.ShapeDtypeStruct((1, _H), jnp.float32)]
        out_specs += [vblk, row1]

    f = pl.pallas_call(body, grid=grid, in_specs=in_specs,
                       out_specs=out_specs, out_shape=out_shape)
    args = list(xin)
    for W, av in srcs:
        args += [W, av]
    for WT, av in dsts:
        args += [WT, av]
    return f(*args)


def _add2(a, b):
    n = a.shape[0]
    blk = pl.BlockSpec((_BM, _H), lambda i: (i, 0))

    def body(ar, br, orf):
        orf[...] = ar[...] + br[...]

    return pl.pallas_call(body, grid=(pl.cdiv(n, _BM),),
                          in_specs=[blk, blk], out_specs=blk,
                          out_shape=jax.ShapeDtypeStruct((n, _H), jnp.float32))(a, b)


_RCPAD = ((_MAXRC + _B - 1) // _B) * _B


@functools.cache
def _make_agg(n_src, n_dst):
    chunks = _dst_chunks(n_dst)
    mesh = plsc.VectorSubcoreMesh(core_axis_name="c", subcore_axis_name="s")

    @functools.partial(
        pl.kernel, mesh=mesh,
        compiler_params=pltpu.CompilerParams(needs_layout_passes=False),
        out_type=jax.ShapeDtypeStruct((n_dst, _H), jnp.float32),
        scratch_types=[
            pltpu.VMEM_SHARED((_RCPAD, _H), jnp.float32),
            pltpu.VMEM_SHARED((16 * _RCPAD,), jnp.float32),
            pltpu.VMEM_SHARED((_RCPAD,), jnp.float32),
            pltpu.VMEM((_ET + 16,), jnp.int32),
            pltpu.VMEM((_ET + 16,), jnp.int32),
            pltpu.VMEM((_ET + 16,), jnp.float32),
            pltpu.VMEM((_ET + 144,), jnp.int32),
            pltpu.VMEM((_ET + 144,), jnp.int32),
            pltpu.VMEM((_ET + 144,), jnp.float32),
            pltpu.VMEM((_B,), jnp.int32),
            pltpu.VMEM((_B,), jnp.int32),
            pltpu.VMEM((_RCPAD,), jnp.float32),
            pltpu.VMEM((_B, _H), jnp.float32),
            pltpu.VMEM((_B, _H), jnp.float32),
            pltpu.VMEM((_B,), jnp.float32),
            pltpu.VMEM((_B,), jnp.float32),
            pltpu.VMEM((_RCPAD // 16,), jnp.float32),
            pltpu.VMEM((_H,), jnp.float32),
            pltpu.VMEM((_H,), jnp.float32),
            pltpu.VMEM((_H,), jnp.float32),
            pltpu.SemaphoreType.DMA,
            pltpu.SemaphoreType.DMA,
        ],
    )
    def agg(hs, asv, adv, mxs, mxd, bias, srcp, dstp, out,
            spacc, spdenp, spdenF, src_b, dst_b, ex_b,
            srcm_b, dstl_b, exm_b, six, dix, denp, hsb, msgb,
            den_s, den_b, den_r, bias_b, ms_b, md_b, sem, sem2):
        c = lax.axis_index("c")
        s = lax.axis_index("s")
        lanes = lax.broadcasted_iota(jnp.int32, (16,), 0)
        dnums = lax.GatherDimensionNumbers(
            offset_dims=(), collapsed_slice_dims=(0,), start_index_map=(0,))

        def lane_bcast(v, j):
            idx = jnp.full((16, 1), j, jnp.int32)
            return lax.gather(v, idx, dnums, slice_sizes=(1,),
                              mode=lax.GatherScatterMode.PROMISE_IN_BOUNDS)
        zf = jnp.zeros((16,), jnp.float32)
        zi = jnp.zeros((16,), jnp.int32)

        e0 = s * _ET
        pltpu.sync_copy(srcp.at[pl.ds(e0, _ET)], src_b.at[pl.ds(0, _ET)])
        pltpu.sync_copy(dstp.at[pl.ds(e0, _ET)], dst_b.at[pl.ds(0, _ET)])
        pltpu.sync_copy(mxs.at[0], ms_b)
        pltpu.sync_copy(mxd.at[0], md_b)
        pltpu.sync_copy(bias, bias_b)
        pltpu.async_copy(asv.at[src_b.at[pl.ds(0, _ET)]], ex_b.at[pl.ds(0, _ET)], sem)
        pltpu.async_copy(adv.at[dst_b.at[pl.ds(0, _ET)]], exm_b.at[pl.ds(0, _ET)], sem2)
        pltpu.make_async_copy(asv.at[src_b.at[pl.ds(0, _ET)]], ex_b.at[pl.ds(0, _ET)], sem).wait()
        pltpu.make_async_copy(adv.at[dst_b.at[pl.ds(0, _ET)]], exm_b.at[pl.ds(0, _ET)], sem2).wait()

        msum = ms_b[pl.ds(0, 16)] + md_b[pl.ds(0, 16)]
        cshift = jnp.where(msum > 0, msum, 0.2 * msum)
        bvecs = [bias_b[pl.ds(16 * k, 16)] for k in range(8)]

        def ex_body(i, _):
            a = ex_b[pl.ds(i * 16, 16)] + exm_b[pl.ds(i * 16, 16)]
            al = jnp.where(a > 0, a, 0.2 * a)
            gi = e0 + i * 16 + lanes
            ex_b[pl.ds(i * 16, 16)] = jnp.where(gi < _E, jnp.exp(al - cshift), 0.0)
            return 0

        lax.fori_loop(0, _ET // 16, ex_body, 0)

        def chunk_pass(cb, csz):
            nfull, prem = csz // _B, csz % _B
            nround = (nfull + 15) // 16
            if True:
                def mz_body(i, _):
                    msgb[i // 8, pl.ds((i % 8) * 16, 16)] = zf
                    return 0

                lax.fori_loop(0, _B * 8, mz_body, 0)

                def zero_blk(t, _):
                    z = t * 16 + s

                    @pl.when(z < nfull)
                    def _():
                        pltpu.sync_copy(msgb, spacc.at[pl.ds(z * _B, _B)])
                    return 0

                lax.fori_loop(0, nround, zero_blk, 0)
                if prem:
                    @pl.when(s == nfull % 16)
                    def _():
                        pltpu.sync_copy(msgb.at[pl.ds(0, prem)],
                                        spacc.at[pl.ds(nfull * _B, prem)])

                def dz_body(i, _):
                    for u in range(4):
                        denp[pl.ds(i * 64 + u * 16, 16)] = zf
                    return 0

                lax.fori_loop(0, _RCPAD // 64, dz_body, 0)
                plsc.subcore_barrier()

                def scan_one(i, cnt):
                    d = dst_b[pl.ds(i * 16, 16)]
                    ex = ex_b[pl.ds(i * 16, 16)]
                    m = (d >= cb) & (d < cb + csz)
                    dl = d - cb
                    plsc.addupdate_scatter(denp, [dl], ex, mask=m)
                    plsc.store_compressed(srcm_b.at[pl.ds(cnt, 16)],
                                          src_b[pl.ds(i * 16, 16)], mask=m)
                    plsc.store_compressed(dstl_b.at[pl.ds(cnt, 16)], dl, mask=m)
                    plsc.store_compressed(exm_b.at[pl.ds(cnt, 16)], ex, mask=m)
                    return cnt + jnp.sum(m.astype(jnp.int32))

                def scan_body(i, cnt):
                    for u in range(4):
                        cnt = scan_one(i * 4 + u, cnt)
                    return cnt

                nsc4 = (_ET // 16) // 4
                cnt = lax.fori_loop(0, nsc4, scan_body, jnp.int32(0))
                for u in range(nsc4 * 4, _ET // 16):
                    cnt = scan_one(u, cnt)
                pltpu.sync_copy(denp, spdenp.at[pl.ds(s * _RCPAD, _RCPAD)])

                def pad_body(k, _):
                    srcm_b[pl.ds(cnt + k * 16, 16)] = zi
                    dstl_b[pl.ds(cnt + k * 16, 16)] = zi
                    exm_b[pl.ds(cnt + k * 16, 16)] = zf
                    return 0

                lax.fori_loop(0, _B // 16, pad_body, 0)
                nb = (cnt + _B - 1) // _B

                def batch_body(bi, _):
                    o = bi * _B
                    for k in range(_B // 16):
                        six[pl.ds(k * 16, 16)] = srcm_b[pl.ds(o + k * 16, 16)]
                        dix[pl.ds(k * 16, 16)] = dstl_b[pl.ds(o + k * 16, 16)]
                    pltpu.async_copy(hs.at[six], hsb, sem).wait()

                    for g in range(_B // 16):
                        exv = exm_b[pl.ds(o + g * 16, 16)]
                        for j in range(16):
                            exsp = lane_bcast(exv, j)
                            r = g * 16 + j
                            for k in range(8):
                                msgb[r, pl.ds(k * 16, 16)] = (
                                    hsb[r, pl.ds(k * 16, 16)] * exsp)
                    pltpu.sync_copy(msgb, spacc.at[dix], add=True)
                    return 0

                lax.fori_loop(0, nb, batch_body, 0)
                plsc.subcore_barrier()

                rs = _RCPAD // 16
                d0 = s * rs

                def dred(t, _):
                    pltpu.sync_copy(spdenp.at[pl.ds(t * _RCPAD + d0, rs)], den_r)

                    def dacc(j, _):
                        cur = den_r[pl.ds(j * 16, 16)]
                        prev = denp[pl.ds(j * 16, 16)]
                        denp[pl.ds(j * 16, 16)] = jnp.where(t == 0, cur, prev + cur)
                        return 0

                    lax.fori_loop(0, rs // 16, dacc, 0)
                    return 0

                lax.fori_loop(0, 16, dred, 0)
                pltpu.sync_copy(denp.at[pl.ds(0, rs)], spdenF.at[pl.ds(d0, rs)])
                plsc.subcore_barrier()

                def flush_blk(z, rows):
                    pltpu.sync_copy(spacc.at[pl.ds(z * _B, rows)],
                                    hsb.at[pl.ds(0, rows)])
                    pltpu.sync_copy(spdenF.at[pl.ds(z * _B, rows)],
                                    den_s.at[pl.ds(0, rows)])
                    for k in range(rows // 16):
                        den_b[pl.ds(k * 16, 16)] = (
                            1.0 / (den_s[pl.ds(k * 16, 16)] + 1e-16))

                    for g in range(rows // 16):
                        rcv = den_b[pl.ds(g * 16, 16)]
                        for j in range(16):
                            rc = lane_bcast(rcv, j)
                            r = g * 16 + j
                            for k in range(8):
                                msgb[r, pl.ds(k * 16, 16)] = (
                                    hsb[r, pl.ds(k * 16, 16)] * rc + bvecs[k])
                    pltpu.sync_copy(msgb.at[pl.ds(0, rows)],
                                    out.at[pl.ds(cb + z * _B, rows)])

                def flush_round(t, _):
                    z = t * 16 + s

                    @pl.when(z < nfull)
                    def _():
                        flush_blk(z, _B)
                    return 0

                lax.fori_loop(0, nround, flush_round, 0)
                if prem:
                    @pl.when(s == nfull % 16)
                    def _():
                        flush_blk(jnp.int32(nfull), prem)
                plsc.subcore_barrier()

        nu = len(chunks) - 1
        ubase = chunks[0][1]
        nmy = (nu - c + 1) // 2

        def uni_body(t, _):
            cb = pl.multiple_of((2 * t + c) * ubase, 16)
            chunk_pass(cb, ubase)
            return 0

        lax.fori_loop(0, nmy, uni_body, 0)

        @pl.when(c == (nu % 2))
        def _():
            chunk_pass(chunks[-1][0], chunks[-1][1])

    return agg


def kernel(x_user, x_product, x_review, params,
           edge_index_writes, edge_index_reviews,
           edge_index_rev_by, edge_index_written_by):
    pads = _EPAD - _E

    def prep_edges(ei):
        return (jnp.pad(ei[0], (0, pads)), jnp.pad(ei[1], (0, pads)))

    e_w = prep_edges(edge_index_writes)
    e_rv = prep_edges(edge_index_reviews)
    e_rb = prep_edges(edge_index_rev_by)
    e_wb = prep_edges(edge_index_written_by)

    agg_u2r = _make_agg(_NU, _NR)
    agg_p2r = _make_agg(_NP, _NR)
    agg_r2p = _make_agg(_NR, _NP)
    agg_r2u = _make_agg(_NR, _NU)

    def srcp(cp):
        return (cp['Ws'], cp['as'][None])

    def dstp(cp):
        return (cp['Wd'].T, cp['ad'][None])

    def layer(cp, mode_u, xin_u, mode_p, xin_p, mode_r, xin_r):
        hs_w, as_w, ms_w, ad_wb, md_wb = _node_stage(
            _NU, mode_u, xin_u, [srcp(cp['writes'])], [dstp(cp['written_by'])])
        hs_rb, as_rb, ms_rb, ad_rv, md_rv = _node_stage(
            _NP, mode_p, xin_p, [srcp(cp['rev_by'])], [dstp(cp['reviews'])])
        (hs_rv, as_rv, ms_rv, hs_wb, as_wb, ms_wb,
         ad_w, md_w, ad_rb, md_rb) = _node_stage(
            _NR, mode_r, xin_r,
            [srcp(cp['reviews']), srcp(cp['written_by'])],
            [dstp(cp['writes']), dstp(cp['rev_by'])])

        o_w = agg_u2r(hs_w, as_w, ad_w, ms_w, md_w, cp['writes']['b'], *e_w)
        o_rb = agg_p2r(hs_rb, as_rb, ad_rb, ms_rb, md_rb, cp['rev_by']['b'], *e_rb)
        o_rv = agg_r2p(hs_rv, as_rv, ad_rv, ms_rv, md_rv, cp['reviews']['b'], *e_rv)
        o_wb = agg_r2u(hs_wb, as_wb, ad_wb, ms_wb, md_wb, cp['written_by']['b'], *e_wb)
        return o_w, o_rb, o_rv, o_wb

    o_w1, o_rb1, o_rv1, o_wb1 = layer(
        params['conv1'],
        'lin', (x_user, params['lin_user_W'], params['lin_user_b'][None]),
        'lin', (x_product, params['lin_product_W'], params['lin_product_b'][None]),
        'lin', (x_review, params['lin_review_W'], params['lin_review_b'][None]))

    o_w2, o_rb2, o_rv2, o_wb2 = layer(
        params['conv2'],
        'relu1', (o_wb1,), 'relu1', (o_rv1,), 'relu2', (o_w1, o_rb1))

    return o_wb2, o_rv2, _add2(o_w2, o_rb2)
